# baseline (speedup 1.0000x reference)
import functools

import jax
import jax.numpy as jnp
from jax import lax
from jax.experimental import pallas as pl
from jax.experimental.pallas import tpu as pltpu
from jax.experimental.pallas import tpu_sc as plsc

_BATCH = 4096
_HIST = 200
_DIM = 64
_NW = 32
_BBLK = _BATCH // _NW
_L = 16


def _transpose_chunk(rows_v, tr_st, tr_nst):
    lanes = lax.iota(jnp.int32, _L)

    def diag(t, _):
        colsel = lax.bitwise_and(lanes + t, _L - 1)
        for dblk in range(8):
            tr = tr_st if dblk < 4 else tr_nst
            dbase = (dblk % 4) * _L
            col = dblk * _L + colsel
            trrow = dbase + colsel
            for jblk in range(8):
                row = jblk * _L + lanes
                v = plsc.load_gather(rows_v, [row, col])
                plsc.store_scatter(tr, [trrow, row], v)
        return _

    lax.fori_loop(0, _L, diag, 0)


def _make_kernel():
    mesh = plsc.VectorSubcoreMesh(core_axis_name="c", subcore_axis_name="s")

    @functools.partial(
        pl.kernel,
        mesh=mesh,
        compiler_params=pltpu.CompilerParams(needs_layout_passes=False),
        out_type=(
            jax.ShapeDtypeStruct((_HIST, _DIM, _BATCH), jnp.float32),
            jax.ShapeDtypeStruct((_HIST, _DIM, _BATCH), jnp.float32),
        ),
        scratch_types=[
            pltpu.VMEM((_HIST, _BBLK), jnp.int32),
            pltpu.VMEM((_BBLK, 2 * _DIM), jnp.float32),
            pltpu.VMEM((_BBLK, 2 * _DIM), jnp.float32),
            pltpu.VMEM((_DIM, _BBLK), jnp.float32),
            pltpu.VMEM((_DIM, _BBLK), jnp.float32),
            pltpu.VMEM((_DIM, _BBLK), jnp.float32),
            pltpu.VMEM((_DIM, _BBLK), jnp.float32),
            pltpu.SemaphoreType.DMA,
            pltpu.SemaphoreType.DMA,
            pltpu.SemaphoreType.DMA,
            pltpu.SemaphoreType.DMA,
        ],
    )
    def dual_gather(idx_hbm, tab_hbm, out_st, out_nst,
                    idx_v, rows_a, rows_b, tra_st, tra_nst, trb_st, trb_nst,
                    sem_a, sem_b, sem_wa, sem_wb):
        wid = lax.axis_index("s") * 2 + lax.axis_index("c")
        b0 = wid * _BBLK
        pltpu.sync_copy(idx_hbm.at[wid], idx_v)

        pltpu.async_copy(tab_hbm.at[idx_v.at[0]], rows_a, sem_a)
        pltpu.async_copy(tab_hbm.at[idx_v.at[1]], rows_b, sem_b)

        def process(h, i, rows, sem, t_st, t_nst, sem_w):
            pltpu.make_async_copy(tab_hbm.at[idx_v.at[h]], rows, sem).wait()

            @pl.when(i > 0)
            def _():
                pltpu.make_async_copy(
                    t_st, out_st.at[h, :, pl.ds(b0, _BBLK)], sem_w).wait()
                pltpu.make_async_copy(
                    t_nst, out_nst.at[h, :, pl.ds(b0, _BBLK)], sem_w).wait()

            _transpose_chunk(rows, t_st, t_nst)

            @pl.when(h + 2 < _HIST)
            def _():
                pltpu.async_copy(tab_hbm.at[idx_v.at[h + 2]], rows, sem)

            pltpu.async_copy(t_st, out_st.at[h, :, pl.ds(b0, _BBLK)], sem_w)
            pltpu.async_copy(t_nst, out_nst.at[h, :, pl.ds(b0, _BBLK)], sem_w)

        def step(i, carry):
            process(2 * i, i, rows_a, sem_a, tra_st, tra_nst, sem_wa)
            process(2 * i + 1, i, rows_b, sem_b, trb_st, trb_nst, sem_wb)
            return carry

        lax.fori_loop(0, _HIST // 2, step, 0)

        for t_st, t_nst, sem_w in ((tra_st, tra_nst, sem_wa),
                                   (trb_st, trb_nst, sem_wb)):
            pltpu.make_async_copy(
                t_st, out_st.at[0, :, pl.ds(b0, _BBLK)], sem_w).wait()
            pltpu.make_async_copy(
                t_nst, out_nst.at[0, :, pl.ds(b0, _BBLK)], sem_w).wait()

    return dual_gather


_DUAL_GATHER = _make_kernel()


@jax.jit
def kernel(inputs, static_table, non_static_table):
    idx = inputs.T.reshape(_HIST, _NW, _BBLK).transpose(1, 0, 2)
    fused = jnp.concatenate([static_table, non_static_table], axis=1)
    out_st, out_nst = _DUAL_GATHER(idx, fused)
    return (out_st.transpose(2, 0, 1), out_nst.transpose(2, 0, 1))


# speedup vs baseline: 2.7515x; 2.7515x over previous
import functools

import jax
import jax.numpy as jnp
from jax import lax
from jax.experimental import pallas as pl
from jax.experimental.pallas import tpu as pltpu
from jax.experimental.pallas import tpu_sc as plsc

_BATCH = 4096
_HIST = 200
_DIM = 64
_NW = 32
_BBLK = _BATCH // _NW
_L = 16


def _transpose_chunk(rows_v, tr_st, tr_nst):
    lanes = lax.iota(jnp.int32, _L)
    colsel = [lax.bitwise_and(lanes + t, _L - 1) for t in range(_L)]

    def jstep(jblk, carry):
        row = jblk * _L + lanes
        for dblk in range(8):
            tr = tr_st if dblk < 4 else tr_nst
            dbase = (dblk % 4) * _L
            vals = [plsc.load_gather(rows_v, [row, dblk * _L + colsel[t]])
                    for t in range(_L)]
            for t in range(_L):
                plsc.store_scatter(tr, [dbase + colsel[t], row], vals[t])
        return carry

    lax.fori_loop(0, 8, jstep, 0)


_VOCAB = 100000
_VB = _VOCAB // 128


def _transpose_tile64(src, comb, cbase):
    lanes = lax.iota(jnp.int32, _L)
    colsel = [lax.bitwise_and(lanes + t, _L - 1) for t in range(_L)]

    def istep(igrp, carry):
        i0 = igrp * _L
        for dgrp in range(4):
            drow = dgrp * _L + lanes
            icol = [i0 + colsel[t] for t in range(_L)]
            vals = [plsc.load_gather(src, [drow, icol[t]])
                    for t in range(_L)]
            for t in range(_L):
                plsc.store_scatter(comb, [icol[t], cbase + drow], vals[t])
        return carry

    lax.fori_loop(0, 8, istep, 0)


def _make_fuse_kernel():
    mesh = plsc.VectorSubcoreMesh(core_axis_name="c", subcore_axis_name="s")

    @functools.partial(
        pl.kernel,
        mesh=mesh,
        compiler_params=pltpu.CompilerParams(needs_layout_passes=False),
        out_type=jax.ShapeDtypeStruct((_VOCAB, 2 * _DIM), jnp.float32),
        scratch_types=[
            pltpu.VMEM((2, 2, _DIM, 2 * _DIM), jnp.float32),
            pltpu.VMEM((2, 2 * _DIM, 2 * _DIM), jnp.float32),
            pltpu.VMEM((_VOCAB - _VB * 2 * _DIM, 2 * _DIM), jnp.float32),
            pltpu.SemaphoreType.DMA,
            pltpu.SemaphoreType.DMA,
            pltpu.SemaphoreType.DMA,
            pltpu.SemaphoreType.DMA,
        ],
    )
    def fuse(st_hbm, nst_hbm, tail_hbm, fused, ab, comb, tail_v,
             sem_a, sem_b, sem_w0, sem_w1):
        wid = lax.axis_index("s") * 2 + lax.axis_index("c")
        sems = (sem_a, sem_b)
        wsems = (sem_w0, sem_w1)

        @pl.when(wid == 0)
        def _():
            pltpu.sync_copy(tail_hbm, tail_v)
            pltpu.sync_copy(tail_v, fused.at[pl.ds(_VB * 2 * _DIM,
                                                   _VOCAB - _VB * 2 * _DIM)])

        def v0_of(j):
            return (wid + _NW * j) * 2 * _DIM

        def fetch(j, slot):
            v0 = v0_of(j)
            pltpu.async_copy(st_hbm.at[:, pl.ds(v0, 2 * _DIM)],
                             ab.at[slot, 0], sems[slot])
            pltpu.async_copy(nst_hbm.at[:, pl.ds(v0, 2 * _DIM)],
                             ab.at[slot, 1], sems[slot])

        fetch(0, 0)

        def step(j, carry):
            slot = lax.rem(j, 2)
            for s in range(2):
                @pl.when(slot == s)
                def _():
                    v0 = v0_of(j)
                    pltpu.make_async_copy(
                        st_hbm.at[:, pl.ds(v0, 2 * _DIM)],
                        ab.at[s, 0], sems[s]).wait()
                    pltpu.make_async_copy(
                        nst_hbm.at[:, pl.ds(v0, 2 * _DIM)],
                        ab.at[s, 1], sems[s]).wait()

                    @pl.when(wid + _NW * (j + 1) < _VB)
                    def _():
                        fetch(j + 1, 1 - s)

                    @pl.when(j >= 2)
                    def _():
                        pltpu.make_async_copy(
                            comb.at[s], fused.at[pl.ds(0, 2 * _DIM)],
                            wsems[s]).wait()

                    _transpose_tile64(ab.at[s, 0], comb.at[s], 0)
                    _transpose_tile64(ab.at[s, 1], comb.at[s], _DIM)
                    pltpu.async_copy(comb.at[s],
                                     fused.at[pl.ds(v0, 2 * _DIM)], wsems[s])
            return carry

        nblk = lax.div(jnp.int32(_VB - 1 - wid), jnp.int32(_NW)) + 1
        lax.fori_loop(0, nblk, step, 0)

        for s in range(2):
            pltpu.make_async_copy(
                comb.at[s], fused.at[pl.ds(0, 2 * _DIM)], wsems[s]).wait()

    return fuse


def _make_kernel():
    mesh = plsc.VectorSubcoreMesh(core_axis_name="c", subcore_axis_name="s")

    @functools.partial(
        pl.kernel,
        mesh=mesh,
        compiler_params=pltpu.CompilerParams(needs_layout_passes=False),
        out_type=(
            jax.ShapeDtypeStruct((_HIST, _DIM, _BATCH), jnp.float32),
            jax.ShapeDtypeStruct((_HIST, _DIM, _BATCH), jnp.float32),
        ),
        scratch_types=[
            pltpu.VMEM((_HIST, _BBLK), jnp.int32),
            pltpu.VMEM((4, _BBLK, 2 * _DIM), jnp.float32),
            pltpu.VMEM((_DIM, _BBLK), jnp.float32),
            pltpu.VMEM((_DIM, _BBLK), jnp.float32),
            pltpu.VMEM((_DIM, _BBLK), jnp.float32),
            pltpu.VMEM((_DIM, _BBLK), jnp.float32),
            pltpu.SemaphoreType.DMA,
            pltpu.SemaphoreType.DMA,
            pltpu.SemaphoreType.DMA,
            pltpu.SemaphoreType.DMA,
            pltpu.SemaphoreType.DMA,
            pltpu.SemaphoreType.DMA,
        ],
    )
    def dual_gather(idx_hbm, tab_hbm, out_st, out_nst,
                    idx_v, rows4, tra_st, tra_nst, trb_st, trb_nst,
                    sem_r0, sem_r1, sem_r2, sem_r3, sem_wa, sem_wb):
        wid = lax.axis_index("s") * 2 + lax.axis_index("c")
        b0 = wid * _BBLK
        sem_r = (sem_r0, sem_r1, sem_r2, sem_r3)
        trs = ((tra_st, tra_nst, sem_wa), (trb_st, trb_nst, sem_wb))
        pltpu.sync_copy(idx_hbm.at[wid], idx_v)

        pltpu.async_copy(tab_hbm.at[idx_v.at[0]], rows4.at[0], sem_r0)
        pltpu.async_copy(tab_hbm.at[idx_v.at[1]], rows4.at[1], sem_r1)

        def step(i, carry):
            for k in range(4):
                h = 4 * i + k
                rows = rows4.at[k]
                t_st, t_nst, sem_w = trs[k % 2]
                pltpu.make_async_copy(
                    tab_hbm.at[idx_v.at[h]], rows, sem_r[k]).wait()

                @pl.when(h + 2 < _HIST)
                def _():
                    pltpu.async_copy(tab_hbm.at[idx_v.at[h + 2]],
                                     rows4.at[(k + 2) % 4], sem_r[(k + 2) % 4])

                @pl.when(h >= 2)
                def _():
                    pltpu.make_async_copy(
                        t_st, out_st.at[h, :, pl.ds(b0, _BBLK)], sem_w).wait()
                    pltpu.make_async_copy(
                        t_nst, out_nst.at[h, :, pl.ds(b0, _BBLK)], sem_w).wait()

                _transpose_chunk(rows, t_st, t_nst)

                pltpu.async_copy(t_st, out_st.at[h, :, pl.ds(b0, _BBLK)], sem_w)
                pltpu.async_copy(t_nst, out_nst.at[h, :, pl.ds(b0, _BBLK)], sem_w)
            return carry

        lax.fori_loop(0, _HIST // 4, step, 0)

        for t_st, t_nst, sem_w in trs:
            pltpu.make_async_copy(
                t_st, out_st.at[0, :, pl.ds(b0, _BBLK)], sem_w).wait()
            pltpu.make_async_copy(
                t_nst, out_nst.at[0, :, pl.ds(b0, _BBLK)], sem_w).wait()

    return dual_gather


_FUSE = _make_fuse_kernel()
_DUAL_GATHER = _make_kernel()


@jax.jit
def kernel(inputs, static_table, non_static_table):
    idx = inputs.T.reshape(_HIST, _NW, _BBLK).transpose(1, 0, 2)
    tail = jnp.concatenate([static_table[_VB * 2 * _DIM:],
                            non_static_table[_VB * 2 * _DIM:]], axis=1)
    fused = _FUSE(static_table.T, non_static_table.T, tail)
    out_st, out_nst = _DUAL_GATHER(idx, fused)
    return (out_st.transpose(2, 0, 1), out_nst.transpose(2, 0, 1))
